# baseline (speedup 1.0000x reference)
import functools

import jax
import jax.numpy as jnp
from jax import lax
from jax.experimental import pallas as pl
from jax.experimental.pallas import tpu as pltpu
from jax.experimental.pallas import tpu_sc as plsc

VOCAB = 100000
DIM = 64
BATCH = 4096
HIST = 50
EPS = 1e-5

NC = 2
NS = 16
NW = NC * NS

B_PER_W = BATCH // NW
NB = 2

MAXNORM = 1.0 - EPS
MAXNORM2 = MAXNORM * MAXNORM


def _build():
    mesh = plsc.VectorSubcoreMesh(core_axis_name="c", subcore_axis_name="s")

    @functools.partial(
        pl.kernel,
        mesh=mesh,
        out_type=jax.ShapeDtypeStruct(
            (HIST, DIM // 8, BATCH // 128, 8, 128), jnp.float32
        ),
        scratch_types=[
            pltpu.VMEM((HIST, B_PER_W), jnp.int32),
            pltpu.VMEM((NB, B_PER_W, DIM), jnp.float32),
            pltpu.VMEM((NB, DIM // 8, 8, B_PER_W + 1), jnp.float32),
        ] + [pltpu.SemaphoreType.DMA] * 4,
        compiler_params=pltpu.CompilerParams(
            use_tc_tiling_on_sc=False, needs_layout_passes=False
        ),
    )
    def body(table_hbm, ext_hbm, out_hbm, idx_v, inb, outb, g0, g1, s0, s1):
        wid = lax.axis_index("s") * NC + lax.axis_index("c")
        b_base = wid * B_PER_W
        pltpu.sync_copy(ext_hbm.at[:, pl.ds(b_base, B_PER_W)], idx_v)
        gsems = [g0, g1]
        ssems = [s0, s1]

        lanes = lax.iota(jnp.int32, 16)
        perms = [lanes ^ shift for shift in (8, 4, 2, 1)]
        dvecs = [lanes + 16 * k for k in range(4)]
        tr_vecs = [lax.shift_right_logical(d, 3) for d in dvecs]
        rc_vecs = [lax.bitwise_and(d, jnp.int32(7)) for d in dvecs]

        def gather(h, b):
            return pltpu.make_async_copy(
                table_hbm.at[idx_v.at[h]], inb.at[b], gsems[b]
            )

        def scatter(h, b):
            return pltpu.make_async_copy(
                outb.at[b, :, :, pl.ds(0, B_PER_W)],
                out_hbm.at[h, :, wid, :, :],
                ssems[b],
            )

        for b in range(NB):
            gather(jnp.int32(b), b).start()

        def outer(g, carry):
            for b in range(NB):
                h = g * NB + b
                gather(h, b).wait()

                @pl.when(g > 0)
                def _():
                    scatter(jnp.int32(0), b).wait()

                src = inb.at[b]
                dst = outb.at[b]

                @plsc.parallel_loop(0, B_PER_W, unroll=8)
                def _(r):
                    v0 = src[r, pl.ds(0, 16)]
                    v1 = src[r, pl.ds(16, 16)]
                    v2 = src[r, pl.ds(32, 16)]
                    v3 = src[r, pl.ds(48, 16)]
                    x = v0 * v0 + v1 * v1 + v2 * v2 + v3 * v3
                    for p in perms:
                        x = x + x.at[p].get(mode="promise_in_bounds")
                    i = lax.bitcast_convert_type(x, jnp.int32)
                    i = jnp.int32(0x5F3759DF) - lax.shift_right_logical(i, 1)
                    y = lax.bitcast_convert_type(i, jnp.float32)
                    for _ in range(3):
                        y = y * (1.5 - 0.5 * x * y * y)
                    scale = jnp.where(
                        x > MAXNORM2,
                        MAXNORM * y,
                        jnp.full((16,), 1.0, dtype=jnp.float32),
                    )
                    col = jnp.full((16,), r, dtype=jnp.int32)
                    vs = [v0 * scale, v1 * scale, v2 * scale, v3 * scale]
                    for k in range(4):
                        plsc.store_scatter(
                            dst, [tr_vecs[k], rc_vecs[k], col], vs[k]
                        )

                scatter(h, b).start()

                @pl.when(h + NB < HIST)
                def _():
                    gather(h + NB, b).start()

            return carry

        lax.fori_loop(0, HIST // NB, outer, 0)
        for b in range(NB):
            scatter(jnp.int32(0), b).wait()

    return body


_sc_lookup = _build()


def kernel(examples, table):
    out5 = _sc_lookup(table, examples.T)
    return jnp.transpose(out5, (2, 4, 0, 1, 3)).reshape(BATCH, HIST, DIM)


# speedup vs baseline: 1.0445x; 1.0445x over previous
import functools

import jax
import jax.numpy as jnp
from jax import lax
from jax.experimental import pallas as pl
from jax.experimental.pallas import tpu as pltpu
from jax.experimental.pallas import tpu_sc as plsc

VOCAB = 100000
DIM = 64
BATCH = 4096
HIST = 50
EPS = 1e-5

NC = 2
NS = 16
NW = NC * NS

B_PER_W = BATCH // NW
NB = 2

MAXNORM = 1.0 - EPS
MAXNORM2 = MAXNORM * MAXNORM


def _build():
    mesh = plsc.VectorSubcoreMesh(core_axis_name="c", subcore_axis_name="s")

    @functools.partial(
        pl.kernel,
        mesh=mesh,
        out_type=jax.ShapeDtypeStruct(
            (HIST, DIM // 8, BATCH // 128, 8, 128), jnp.float32
        ),
        scratch_types=[
            pltpu.VMEM((HIST, B_PER_W), jnp.int32),
            pltpu.VMEM((NB, B_PER_W, DIM), jnp.float32),
            pltpu.VMEM((NB, DIM // 8, 8, B_PER_W + 1), jnp.float32),
        ] + [pltpu.SemaphoreType.DMA] * 4,
        compiler_params=pltpu.CompilerParams(
            use_tc_tiling_on_sc=False, needs_layout_passes=False
        ),
    )
    def body(table_hbm, ext_hbm, out_hbm, idx_v, inb, outb, g0, g1, s0, s1):
        wid = lax.axis_index("s") * NC + lax.axis_index("c")
        b_base = wid * B_PER_W
        pltpu.sync_copy(ext_hbm.at[:, pl.ds(b_base, B_PER_W)], idx_v)
        gsems = [g0, g1]
        ssems = [s0, s1]

        lanes = lax.iota(jnp.int32, 16)
        perms = [lanes ^ shift for shift in (8, 4, 2, 1)]
        dvecs = [lanes + 16 * k for k in range(4)]
        tr_vecs = [lax.shift_right_logical(d, 3) for d in dvecs]
        rc_vecs = [lax.bitwise_and(d, jnp.int32(7)) for d in dvecs]

        def gather(h, b):
            return pltpu.make_async_copy(
                table_hbm.at[idx_v.at[h]], inb.at[b], gsems[b]
            )

        def scatter(h, b):
            return pltpu.make_async_copy(
                outb.at[b, :, :, pl.ds(0, B_PER_W)],
                out_hbm.at[h, :, wid, :, :],
                ssems[b],
            )

        for b in range(NB):
            gather(jnp.int32(b), b).start()

        def outer(g, carry):
            for b in range(NB):
                h = g * NB + b
                gather(h, b).wait()

                @pl.when(g > 0)
                def _():
                    scatter(jnp.int32(0), b).wait()

                src = inb.at[b]
                dst = outb.at[b]

                @plsc.parallel_loop(0, B_PER_W, unroll=4)
                def _(r):
                    v0 = src[r, pl.ds(0, 16)]
                    v1 = src[r, pl.ds(16, 16)]
                    v2 = src[r, pl.ds(32, 16)]
                    v3 = src[r, pl.ds(48, 16)]
                    p = v0 * v0 + v1 * v1 + v2 * v2 + v3 * v3
                    x = jnp.full((16,), jnp.sum(p), dtype=jnp.float32)
                    i = lax.bitcast_convert_type(x, jnp.int32)
                    i = jnp.int32(0x5F3759DF) - lax.shift_right_logical(i, 1)
                    y = lax.bitcast_convert_type(i, jnp.float32)
                    for _ in range(2):
                        y = y * (1.5 - 0.5 * x * y * y)
                    scale = jnp.where(
                        x > MAXNORM2,
                        MAXNORM * y,
                        jnp.full((16,), 1.0, dtype=jnp.float32),
                    )
                    col = jnp.full((16,), r, dtype=jnp.int32)
                    vs = [v0 * scale, v1 * scale, v2 * scale, v3 * scale]
                    for k in range(4):
                        plsc.store_scatter(
                            dst, [tr_vecs[k], rc_vecs[k], col], vs[k]
                        )

                scatter(h, b).start()

                @pl.when(h + NB < HIST)
                def _():
                    gather(h + NB, b).start()

            return carry

        lax.fori_loop(0, HIST // NB, outer, 0)
        for b in range(NB):
            scatter(jnp.int32(0), b).wait()

    return body


_sc_lookup = _build()


def kernel(examples, table):
    out5 = _sc_lookup(table, examples.T)
    return jnp.transpose(out5, (2, 4, 0, 1, 3)).reshape(BATCH, HIST, DIM)


# speedup vs baseline: 1.1083x; 1.0611x over previous
import functools

import jax
import jax.numpy as jnp
from jax import lax
from jax.experimental import pallas as pl
from jax.experimental.pallas import tpu as pltpu
from jax.experimental.pallas import tpu_sc as plsc

VOCAB = 100000
DIM = 64
BATCH = 4096
HIST = 50
EPS = 1e-5

NC = 2
NS = 16
NW = NC * NS

B_PER_W = BATCH // NW
NB = 2

MAXNORM = 1.0 - EPS
MAXNORM2 = MAXNORM * MAXNORM


def _build():
    mesh = plsc.VectorSubcoreMesh(core_axis_name="c", subcore_axis_name="s")

    @functools.partial(
        pl.kernel,
        mesh=mesh,
        out_type=jax.ShapeDtypeStruct(
            (HIST, DIM // 8, BATCH // 128, 8, 128), jnp.float32
        ),
        scratch_types=[
            pltpu.VMEM((HIST, B_PER_W), jnp.int32),
            pltpu.VMEM((NB, B_PER_W, DIM), jnp.float32),
            pltpu.VMEM((NB, DIM // 8, 8, B_PER_W + 1), jnp.float32),
        ] + [pltpu.SemaphoreType.DMA] * 4,
        compiler_params=pltpu.CompilerParams(
            use_tc_tiling_on_sc=False, needs_layout_passes=False
        ),
    )
    def body(table_hbm, ext_hbm, out_hbm, idx_v, inb, outb, g0, g1, s0, s1):
        wid = lax.axis_index("s") * NC + lax.axis_index("c")
        b_base = wid * B_PER_W
        pltpu.sync_copy(ext_hbm.at[:, pl.ds(b_base, B_PER_W)], idx_v)
        gsems = [g0, g1]
        ssems = [s0, s1]

        lanes = lax.iota(jnp.int32, 16)
        perms = [lanes ^ shift for shift in (8, 4, 2, 1)]
        dvecs = [lanes + 16 * k for k in range(4)]
        tr_vecs = [lax.shift_right_logical(d, 3) for d in dvecs]
        rc_vecs = [lax.bitwise_and(d, jnp.int32(7)) for d in dvecs]

        def gather(h, b):
            return pltpu.make_async_copy(
                table_hbm.at[idx_v.at[h]], inb.at[b], gsems[b]
            )

        def scatter(h, b):
            return pltpu.make_async_copy(
                outb.at[b, :, :, pl.ds(0, B_PER_W)],
                out_hbm.at[h, :, wid, :, :],
                ssems[b],
            )

        for b in range(NB):
            gather(jnp.int32(b), b).start()

        def outer(g, carry):
            for b in range(NB):
                h = g * NB + b
                gather(h, b).wait()

                @pl.when(g > 0)
                def _():
                    scatter(jnp.int32(0), b).wait()

                src = inb.at[b]
                dst = outb.at[b]

                @plsc.parallel_loop(0, B_PER_W, unroll=4)
                def _(r):
                    v0 = src[r, pl.ds(0, 16)]
                    v1 = src[r, pl.ds(16, 16)]
                    v2 = src[r, pl.ds(32, 16)]
                    v3 = src[r, pl.ds(48, 16)]
                    x = v0 * v0 + v1 * v1 + v2 * v2 + v3 * v3
                    for p in perms:
                        x = x + x.at[p].get(mode="promise_in_bounds")
                    i = lax.bitcast_convert_type(x, jnp.int32)
                    i = jnp.int32(0x5F3759DF) - lax.shift_right_logical(i, 1)
                    y = lax.bitcast_convert_type(i, jnp.float32)
                    for _ in range(2):
                        y = y * (1.5 - 0.5 * x * y * y)
                    scale = jnp.where(
                        x > MAXNORM2,
                        MAXNORM * y,
                        jnp.full((16,), 1.0, dtype=jnp.float32),
                    )
                    col = jnp.full((16,), r, dtype=jnp.int32)
                    vs = [v0 * scale, v1 * scale, v2 * scale, v3 * scale]
                    for k in range(4):
                        plsc.store_scatter(
                            dst, [tr_vecs[k], rc_vecs[k], col], vs[k]
                        )

                scatter(h, b).start()

                @pl.when(h + NB < HIST)
                def _():
                    gather(h + NB, b).start()

            return carry

        lax.fori_loop(0, HIST // NB, outer, 0)
        for b in range(NB):
            scatter(jnp.int32(0), b).wait()

    return body


_sc_lookup = _build()


def kernel(examples, table):
    out5 = _sc_lookup(table, examples.T)
    return jnp.transpose(out5, (2, 4, 0, 1, 3)).reshape(BATCH, HIST, DIM)


# speedup vs baseline: 1.1157x; 1.0067x over previous
import functools

import jax
import jax.numpy as jnp
from jax import lax
from jax.experimental import pallas as pl
from jax.experimental.pallas import tpu as pltpu
from jax.experimental.pallas import tpu_sc as plsc

VOCAB = 100000
DIM = 64
BATCH = 4096
HIST = 50
EPS = 1e-5

NC = 2
NS = 16
NW = NC * NS

B_PER_W = BATCH // NW
NB = 2

MAXNORM = 1.0 - EPS
MAXNORM2 = MAXNORM * MAXNORM


def _build():
    mesh = plsc.VectorSubcoreMesh(core_axis_name="c", subcore_axis_name="s")

    @functools.partial(
        pl.kernel,
        mesh=mesh,
        out_type=jax.ShapeDtypeStruct(
            (HIST, DIM // 8, BATCH // 128, 8, 128), jnp.float32
        ),
        scratch_types=[
            pltpu.VMEM((HIST, B_PER_W), jnp.int32),
            pltpu.VMEM((NB, B_PER_W, DIM), jnp.float32),
            pltpu.VMEM((NB, DIM // 8, 8, B_PER_W + 1), jnp.float32),
        ] + [pltpu.SemaphoreType.DMA] * 4,
        compiler_params=pltpu.CompilerParams(
            use_tc_tiling_on_sc=False, needs_layout_passes=False
        ),
    )
    def body(table_hbm, ext_hbm, out_hbm, idx_v, inb, outb, g0, g1, s0, s1):
        wid = lax.axis_index("s") * NC + lax.axis_index("c")
        b_base = wid * B_PER_W
        pltpu.sync_copy(ext_hbm.at[:, pl.ds(b_base, B_PER_W)], idx_v)
        gsems = [g0, g1]
        ssems = [s0, s1]

        lanes = lax.iota(jnp.int32, 16)
        perms = [lanes ^ shift for shift in (8, 4, 2, 1)]
        dvecs = [lanes + 16 * k for k in range(4)]
        tr_vecs = [lax.shift_right_logical(d, 3) for d in dvecs]
        rc_vecs = [lax.bitwise_and(d, jnp.int32(7)) for d in dvecs]

        def gather(h, b):
            return pltpu.make_async_copy(
                table_hbm.at[idx_v.at[h]], inb.at[b], gsems[b]
            )

        def scatter(h, b):
            return pltpu.make_async_copy(
                outb.at[b, :, :, pl.ds(0, B_PER_W)],
                out_hbm.at[h, :, wid, :, :],
                ssems[b],
            )

        for b in range(NB):
            gather(jnp.int32(b), b).start()

        def outer(g, carry):
            for b in range(NB):
                h = g * NB + b
                gather(h, b).wait()

                @pl.when(g > 0)
                def _():
                    scatter(jnp.int32(0), b).wait()

                src = inb.at[b]
                dst = outb.at[b]

                @plsc.parallel_loop(0, B_PER_W, unroll=4)
                def _(r):
                    v0 = src[r, pl.ds(0, 16)]
                    v1 = src[r, pl.ds(16, 16)]
                    v2 = src[r, pl.ds(32, 16)]
                    v3 = src[r, pl.ds(48, 16)]
                    x = v0 * v0 + v1 * v1 + v2 * v2 + v3 * v3
                    for p in perms:
                        x = x + x.at[p].get(mode="promise_in_bounds")
                    i = lax.bitcast_convert_type(x, jnp.int32)
                    i = jnp.int32(0x5F3759DF) - lax.shift_right_logical(i, 1)
                    y = lax.bitcast_convert_type(i, jnp.float32)
                    for _ in range(3):
                        y = y * (1.5 - 0.5 * x * y * y)
                    scale = jnp.where(
                        x > MAXNORM2,
                        MAXNORM * y,
                        jnp.full((16,), 1.0, dtype=jnp.float32),
                    )
                    col = jnp.full((16,), r, dtype=jnp.int32)
                    vs = [v0 * scale, v1 * scale, v2 * scale, v3 * scale]
                    for k in range(4):
                        plsc.store_scatter(
                            dst, [tr_vecs[k], rc_vecs[k], col], vs[k]
                        )

                scatter(h, b).start()

                @pl.when(h + NB < HIST)
                def _():
                    gather(h + NB, b).start()

            return carry

        lax.fori_loop(0, HIST // NB, outer, 0)
        for b in range(NB):
            scatter(jnp.int32(0), b).wait()

    return body


_sc_lookup = _build()


def kernel(examples, table):
    out5 = _sc_lookup(table, examples.T)
    return jnp.transpose(out5, (2, 4, 0, 1, 3)).reshape(BATCH, HIST, DIM)
